# baseline (speedup 1.0000x reference)
import functools

import jax
import jax.numpy as jnp
from jax import lax
from jax.experimental import pallas as pl
from jax.experimental.pallas import tpu as pltpu
from jax.experimental.pallas import tpu_sc as plsc

SEQ_GRP = 2
NBUF = 3


def _chunks(S):
    out, off = [], 0
    while S - off > 128:
        out.append((off, 104))
        off += 104
    out.append((off, S - off))
    return out


def _make_sc_kernel(B, S, D):
    nc, ns = 2, 16
    nw = nc * ns
    rows_per_w = B // nw
    n_groups = rows_per_w // SEQ_GRP
    chunks = _chunks(S)
    mesh = plsc.VectorSubcoreMesh(core_axis_name="c", subcore_axis_name="s",
                                  num_cores=nc, num_subcores=ns)

    @functools.partial(
        pl.kernel,
        out_type=jax.ShapeDtypeStruct((B, S, D), jnp.float32),
        mesh=mesh,
        scratch_types=[
            pltpu.VMEM((rows_per_w, S), jnp.int32),
            pltpu.VMEM_SHARED((S, D), jnp.float32),
            pltpu.VMEM((NBUF, SEQ_GRP, S, D), jnp.float32),
            [pltpu.SemaphoreType.DMA] * NBUF,
            [pltpu.SemaphoreType.DMA] * NBUF,
            [pltpu.SemaphoreType.DMA] * NBUF,
        ],
        compiler_params=pltpu.CompilerParams(use_tc_tiling_on_sc=False),
    )
    def k(x_hbm, table_hbm, pos_hbm, out_hbm, idx_v, pos_sh, dest,
          sem_pre, sem_gat, sem_wb):
        sid = lax.axis_index("s")
        wid = sid * nc + lax.axis_index("c")
        row0 = wid * rows_per_w
        pltpu.sync_copy(x_hbm.at[pl.ds(row0, rows_per_w)], idx_v)

        @pl.when(sid == 0)
        def _fill_pos():
            pltpu.sync_copy(pos_hbm.at[pl.ds(0, S)], pos_sh)

        plsc.subcore_barrier()

        def gathers(g, b):
            ds = []
            for r in range(SEQ_GRP):
                for off, ln in chunks:
                    ds.append(pltpu.make_async_copy(
                        table_hbm.at[
                            idx_v.at[g * SEQ_GRP + r, pl.ds(off, ln)]],
                        dest.at[b, r, pl.ds(off, ln)],
                        sem_gat[b]))
            return ds

        def prefills(b):
            return [pltpu.make_async_copy(pos_sh, dest.at[b, r], sem_pre[b])
                    for r in range(SEQ_GRP)]

        def writeback(g, b):
            return pltpu.make_async_copy(
                dest.at[b], out_hbm.at[pl.ds(row0 + g * SEQ_GRP, SEQ_GRP)],
                sem_wb[b])

        def step(t, b):
            @pl.when(jnp.logical_and(t >= NBUF, t < n_groups))
            def _wait_wb():
                writeback(t - NBUF, b).wait()

            @pl.when(t < n_groups)
            def _pre():
                for d in prefills(b):
                    d.start()

            bg = (b - 1) % NBUF

            @pl.when(jnp.logical_and(t - 1 >= 0, t - 1 < n_groups))
            def _gat():
                for d in prefills(bg):
                    d.wait()
                for d in gathers(t - 1, bg):
                    d.start(add=True)

            bw = (b - 2) % NBUF

            @pl.when(jnp.logical_and(t - 2 >= 0, t - 2 < n_groups))
            def _wb():
                for d in gathers(t - 2, bw):
                    d.wait()
                writeback(t - 2, bw).start()

        n_steps = n_groups + 2
        n_outer = -(-n_steps // NBUF)

        def outer(i, carry):
            for b in range(NBUF):
                step(i * NBUF + b, b)
            return carry

        lax.fori_loop(0, n_outer, outer, 0)

        for g in range(n_groups - min(NBUF, n_groups), n_groups):
            writeback(g, g % NBUF).wait()

    return k


def kernel(x, emb_table, pos_emb):
    B, S = x.shape
    D = emb_table.shape[1]
    return _make_sc_kernel(B, S, D)(x.astype(jnp.int32), emb_table, pos_emb)
